# baseline (speedup 1.0000x reference)
import jax
import jax.numpy as jnp
from jax.experimental import pallas as pl
from jax.experimental.pallas import tpu as pltpu

BC = 16
BS = 32
C = 96
HW = BC * BS
G = 4
L = G * C
WG = HW // G
BG = BS // G


def _strip_kernel(nact_ref, cols_ref, x_hbm, w_ref, b_ref, m_ref, o_ref,
                  xbuf, sems):
    i = pl.program_id(0)
    slot = jax.lax.rem(i, 2)
    nxt = jax.lax.rem(i + 1, 2)

    def _issue(strip, buf):
        def body(t, _):
            j = cols_ref[strip, t]
            pltpu.make_async_copy(
                x_hbm.at[0, pl.ds(strip * BS, BS), pl.ds(j * BG, BG), :],
                xbuf.at[buf, :, pl.ds(j * BG, BG), :],
                sems.at[buf],
            ).start()
            return 0
        jax.lax.fori_loop(0, nact_ref[strip], body, 0, unroll=False)

    @pl.when(i == 0)
    def _first():
        _issue(0, 0)

    @pl.when(i + 1 < BC)
    def _prefetch():
        _issue(i + 1, nxt)

    def wbody(t, _):
        pltpu.make_async_copy(
            x_hbm.at[0, pl.ds(0, BS), pl.ds(0, BG), :],
            xbuf.at[slot, :, pl.ds(0, BG), :],
            sems.at[slot],
        ).wait()
        return 0
    jax.lax.fori_loop(0, nact_ref[i], wbody, 0, unroll=False)

    x = xbuf[slot].reshape(BS * WG, L)
    y = jnp.dot(x, w_ref[...], preferred_element_type=jnp.float32)
    y = y + b_ref[...]
    y = y.reshape(1, BS, WG, L)
    m = m_ref[...] > 0
    o_ref[...] = jnp.where(m, y, 0.0)


def kernel(inp, active_block_indices, bin_counts, W, b):
    bi = active_block_indices[:, 1]
    bj = active_block_indices[:, 2]
    act2d = jnp.zeros((BC, BC), jnp.int32).at[bi, bj].set(1)
    nact = jnp.sum(act2d, axis=1).astype(jnp.int32)
    cols = jnp.argsort(-act2d, axis=1, stable=True).astype(jnp.int32)
    mask = jnp.repeat(act2d, BG, axis=1).reshape(BC, 1, WG, 1)
    w4 = jnp.kron(jnp.eye(G, dtype=W.dtype), W)
    b4 = jnp.tile(b, (G,)).reshape(1, L)
    x4 = inp.reshape(1, HW, WG, L)

    grid_spec = pltpu.PrefetchScalarGridSpec(
        num_scalar_prefetch=2,
        grid=(BC,),
        in_specs=[
            pl.BlockSpec(memory_space=pl.ANY),
            pl.BlockSpec((L, L), lambda i, *_: (0, 0)),
            pl.BlockSpec((1, L), lambda i, *_: (0, 0)),
            pl.BlockSpec((1, 1, WG, 1), lambda i, *_: (i, 0, 0, 0)),
        ],
        out_specs=pl.BlockSpec((1, BS, WG, L), lambda i, *_: (0, i, 0, 0)),
        scratch_shapes=[
            pltpu.VMEM((2, BS, WG, L), jnp.float32),
            pltpu.SemaphoreType.DMA((2,)),
        ],
    )

    out = pl.pallas_call(
        _strip_kernel,
        grid_spec=grid_spec,
        out_shape=jax.ShapeDtypeStruct((1, HW, WG, L), jnp.float32),
        compiler_params=pltpu.CompilerParams(
            dimension_semantics=("arbitrary",),
        ),
    )(nact, cols, x4, w4, b4, mask)
    return out
